# Initial kernel scaffold; baseline (speedup 1.0000x reference)
import jax
import jax.numpy as jnp
from jax.experimental import pallas as pl


def kernel(edge_index, emb, Wl0, Wr0, b0, Wl1, Wr1, b1):
    raise NotImplementedError("write your pallas kernel here")


# speedup vs baseline: 1.2484x; 1.2484x over previous
import jax
import jax.numpy as jnp
from jax import lax
from jax.experimental import pallas as pl
from jax.experimental.pallas import tpu as pltpu
from jax.experimental.pallas import tpu_sc as plsc

N = 50000
D = 64
H = 64
E = 800000

NC = 2
NS = 16
NW = NC * NS
Q = 8
NPIECE = H // Q
ROW = 128
RPT = 200
NROWS = NW * RPT
EPAD = NROWS * ROW
ZCH = 3136
NPAD = NS * ZCH
BN = 1000


def _sc_piece_body(yp, srcp, dstp, zq_h, outa, outb,
                   src_v, dst_v, rows_v, zv, wbv, acc, sem):
  c = lax.axis_index("c")
  s = lax.axis_index("s")
  w = c * NS + s

  pltpu.sync_copy(srcp.at[pl.ds(w * RPT, RPT)], src_v)
  pltpu.sync_copy(dstp.at[pl.ds(w * RPT, RPT)], dst_v)
  pltpu.sync_copy(zq_h, zv)

  for off in (0, 1568):
    pltpu.sync_copy(zv, acc.at[pl.ds(s * ZCH + off, 1568)])
  plsc.subcore_barrier()

  def step(j, carry):
    pltpu.async_copy(yp.at[src_v.at[j]], rows_v, sem).wait()
    pltpu.sync_copy(rows_v, acc.at[dst_v.at[j]], add=True)
    return carry
  lax.fori_loop(0, RPT, step, 0)

  plsc.subcore_barrier()

  def wb(dst_hbm):
    for off in (0, 1568):
      pltpu.async_copy(acc.at[pl.ds(s * ZCH + off, 1568)], wbv, sem).wait()
      pltpu.async_copy(wbv, dst_hbm.at[pl.ds(s * ZCH + off, 1568)],
                       sem).wait()

  @pl.when(c == 0)
  def _():
    wb(outa)

  @pl.when(c == 1)
  def _():
    wb(outb)


def _make_sc_piece():
  pf32 = jax.ShapeDtypeStruct((NPAD, Q), jnp.float32)
  scratch = [
      pltpu.VMEM((RPT, ROW), jnp.int32),
      pltpu.VMEM((RPT, ROW), jnp.int32),
      pltpu.VMEM((ROW, Q), jnp.float32),
      pltpu.VMEM((1568, Q), jnp.float32),
      pltpu.VMEM((1568, Q), jnp.float32),
      pltpu.VMEM_SHARED((NPAD, Q), jnp.float32),
      pltpu.SemaphoreType.DMA,
  ]
  mesh = plsc.VectorSubcoreMesh(
      core_axis_name="c", subcore_axis_name="s", num_cores=NC, num_subcores=NS)
  return pl.kernel(
      _sc_piece_body,
      out_type=(pf32, pf32),
      mesh=mesh,
      scratch_types=scratch,
      compiler_params=pltpu.CompilerParams(use_tc_tiling_on_sc=False),
  )


def _s1_body(emb_ref, wl_ref, wr_ref, b_ref, y3_ref, z_ref):
  e = emb_ref[...]
  y = jnp.dot(e, wl_ref[...], preferred_element_type=jnp.float32)
  y3_ref[...] = y.reshape(y.shape[0], NPIECE, Q).transpose(1, 0, 2)
  z_ref[...] = jnp.dot(e, wr_ref[...],
                       preferred_element_type=jnp.float32) + b_ref[...]


def _s3_body(aa_ref, ab_ref, deg_ref, z0_ref, wl_ref, wr_ref, bias_ref,
             y3_ref, z_ref, o_ref):
  inv = 1.0 / jnp.maximum(deg_ref[...], 1.0)
  agg3 = aa_ref[...] + ab_ref[...]
  agg = agg3.transpose(1, 0, 2).reshape(agg3.shape[1], H)
  x = agg * inv + z0_ref[...]
  x1 = jnp.maximum(x, 0.0)
  y = jnp.dot(x1, wl_ref[...], preferred_element_type=jnp.float32)
  y3_ref[...] = y.reshape(y.shape[0], NPIECE, Q).transpose(1, 0, 2)
  z_ref[...] = jnp.dot(x1, wr_ref[...],
                       preferred_element_type=jnp.float32) + bias_ref[...]
  nrm = jnp.sqrt(jnp.sum(x * x, axis=1, keepdims=True))
  o_ref[...] = x / jnp.maximum(nrm, 1e-12)


def _tc_stage1(emb, Wl, Wr, b):
  nb = N // BN
  return pl.pallas_call(
      _s1_body,
      grid=(nb,),
      in_specs=[
          pl.BlockSpec((BN, D), lambda i: (i, 0)),
          pl.BlockSpec((D, H), lambda i: (0, 0)),
          pl.BlockSpec((D, H), lambda i: (0, 0)),
          pl.BlockSpec((1, H), lambda i: (0, 0)),
      ],
      out_specs=[
          pl.BlockSpec((NPIECE, BN, Q), lambda i: (0, i, 0)),
          pl.BlockSpec((BN, H), lambda i: (i, 0)),
      ],
      out_shape=[
          jax.ShapeDtypeStruct((NPIECE, N, Q), jnp.float32),
          jax.ShapeDtypeStruct((N, H), jnp.float32),
      ],
  )(emb, Wl, Wr, b)


def _tc_stage3(aggA, aggB, deg, z0, Wl, Wr, b):
  nb = N // BN
  return pl.pallas_call(
      _s3_body,
      grid=(nb,),
      in_specs=[
          pl.BlockSpec((NPIECE, BN, Q), lambda i: (0, i, 0)),
          pl.BlockSpec((NPIECE, BN, Q), lambda i: (0, i, 0)),
          pl.BlockSpec((BN, 1), lambda i: (i, 0)),
          pl.BlockSpec((BN, H), lambda i: (i, 0)),
          pl.BlockSpec((H, H), lambda i: (0, 0)),
          pl.BlockSpec((H, H), lambda i: (0, 0)),
          pl.BlockSpec((1, H), lambda i: (0, 0)),
      ],
      out_specs=[
          pl.BlockSpec((NPIECE, BN, Q), lambda i: (0, i, 0)),
          pl.BlockSpec((BN, H), lambda i: (i, 0)),
          pl.BlockSpec((BN, H), lambda i: (i, 0)),
      ],
      out_shape=[
          jax.ShapeDtypeStruct((NPIECE, N, Q), jnp.float32),
          jax.ShapeDtypeStruct((N, H), jnp.float32),
          jax.ShapeDtypeStruct((N, H), jnp.float32),
      ],
  )(aggA, aggB, deg, z0, Wl, Wr, b)


def kernel(edge_index, emb, Wl0, Wr0, b0, Wl1, Wr1, b1):
  src = edge_index[0]
  dst = edge_index[1]
  srcp = jnp.pad(src, (0, EPAD - E)).reshape(NROWS, ROW)
  dstp = jnp.pad(dst, (0, EPAD - E),
                 constant_values=jnp.int32(N)).reshape(NROWS, ROW)
  zq_h = jnp.zeros((1568, Q), jnp.float32)
  deg = jax.ops.segment_sum(jnp.ones((E,), jnp.float32), dst,
                            num_segments=N).reshape(N, 1)

  sc_piece = _make_sc_piece()
  y3, z0 = _tc_stage1(emb, Wl0, Wr0, b0.reshape(1, H))

  npiece_rt = NPIECE + jnp.minimum(edge_index[0, 1], 0)
  nlayer_rt = 2 + jnp.minimum(edge_index[0, 0], 0)
  zh = jnp.zeros((H, H), jnp.float32)
  zb = jnp.zeros((1, H), jnp.float32)
  wb1 = b1.reshape(1, H)

  def piece_step(k, carry):
    aggA, aggB, y3c = carry
    yp = lax.dynamic_slice(y3c, (k, 0, 0), (1, N, Q)).reshape(N, Q)
    pa, pb = sc_piece(yp, srcp, dstp, zq_h)
    aggA = lax.dynamic_update_slice(aggA, pa[None], (k, 0, 0))
    aggB = lax.dynamic_update_slice(aggB, pb[None], (k, 0, 0))
    return (aggA, aggB, y3c)

  def layer_cond(state):
    return state[0] < nlayer_rt

  def layer_body(state):
    i, y3c, cz, _ = state
    first = i == 0
    wl = jnp.where(first, Wl1, zh)
    wr = jnp.where(first, Wr1, zh)
    bias = jnp.where(first, wb1, zb)
    agg0 = jnp.zeros((NPIECE, NPAD, Q), jnp.float32)
    aggA, aggB, _ = lax.fori_loop(0, npiece_rt, piece_step,
                                  (agg0, agg0, y3c))
    ny3, nz, outn = _tc_stage3(aggA, aggB, deg, cz, wl, wr, bias)
    return (i + 1, ny3, nz, outn)

  state = (jnp.int32(0), y3, z0, jnp.zeros((N, H), jnp.float32))
  state = lax.while_loop(layer_cond, layer_body, state)
  return state[3]


# speedup vs baseline: 1.5704x; 1.2580x over previous
import jax
import jax.numpy as jnp
from jax import lax
from jax.experimental import pallas as pl
from jax.experimental.pallas import tpu as pltpu
from jax.experimental.pallas import tpu_sc as plsc

N = 50000
D = 64
H = 64
E = 800000

NC = 2
NS = 16
NW = NC * NS
Q = 8
NPIECE = H // Q
ROW = 128
RPT = 200
NROWS = NW * RPT
EPAD = NROWS * ROW
ZCH = 3136
NPAD = NS * ZCH
BN = 1000


def _sc_piece_body(yp, srcp, dstp, zq_h, outa, outb,
                   src_v, dst_v, rows_a, rows_b, zv, wbv, acc, sem,
                   gsa, gsb):
  c = lax.axis_index("c")
  s = lax.axis_index("s")
  w = c * NS + s

  pltpu.sync_copy(srcp.at[pl.ds(w * RPT, RPT)], src_v)
  pltpu.sync_copy(dstp.at[pl.ds(w * RPT, RPT)], dst_v)
  pltpu.sync_copy(zq_h, zv)

  for off in (0, 1568):
    pltpu.sync_copy(zv, acc.at[pl.ds(s * ZCH + off, 1568)])
  plsc.subcore_barrier()

  pltpu.async_copy(yp.at[src_v.at[0]], rows_a, gsa)

  def step(j, carry):
    @pl.when(j % 2 == 0)
    def _():
      @pl.when(j < RPT - 1)
      def _():
        pltpu.async_copy(yp.at[src_v.at[j + 1]], rows_b, gsb)
      pltpu.make_async_copy(yp.at[src_v.at[j]], rows_a, gsa).wait()
      pltpu.sync_copy(rows_a, acc.at[dst_v.at[j]], add=True)

    @pl.when(j % 2 == 1)
    def _():
      @pl.when(j < RPT - 1)
      def _():
        pltpu.async_copy(yp.at[src_v.at[j + 1]], rows_a, gsa)
      pltpu.make_async_copy(yp.at[src_v.at[j]], rows_b, gsb).wait()
      pltpu.sync_copy(rows_b, acc.at[dst_v.at[j]], add=True)
    return carry
  lax.fori_loop(0, RPT, step, 0)

  plsc.subcore_barrier()

  def wb(dst_hbm):
    for off in (0, 1568):
      pltpu.async_copy(acc.at[pl.ds(s * ZCH + off, 1568)], wbv, sem).wait()
      pltpu.async_copy(wbv, dst_hbm.at[pl.ds(s * ZCH + off, 1568)],
                       sem).wait()

  @pl.when(c == 0)
  def _():
    wb(outa)

  @pl.when(c == 1)
  def _():
    wb(outb)


def _make_sc_piece():
  pf32 = jax.ShapeDtypeStruct((NPAD, Q), jnp.float32)
  scratch = [
      pltpu.VMEM((RPT, ROW), jnp.int32),
      pltpu.VMEM((RPT, ROW), jnp.int32),
      pltpu.VMEM((ROW, Q), jnp.float32),
      pltpu.VMEM((ROW, Q), jnp.float32),
      pltpu.VMEM((1568, Q), jnp.float32),
      pltpu.VMEM((1568, Q), jnp.float32),
      pltpu.VMEM_SHARED((NPAD, Q), jnp.float32),
      pltpu.SemaphoreType.DMA,
      pltpu.SemaphoreType.DMA,
      pltpu.SemaphoreType.DMA,
  ]
  mesh = plsc.VectorSubcoreMesh(
      core_axis_name="c", subcore_axis_name="s", num_cores=NC, num_subcores=NS)
  return pl.kernel(
      _sc_piece_body,
      out_type=(pf32, pf32),
      mesh=mesh,
      scratch_types=scratch,
      compiler_params=pltpu.CompilerParams(use_tc_tiling_on_sc=False),
  )


def _s1_body(emb_ref, wl_ref, wr_ref, b_ref, y3_ref, z_ref):
  e = emb_ref[...]
  y = jnp.dot(e, wl_ref[...], preferred_element_type=jnp.float32)
  y3_ref[...] = y.reshape(y.shape[0], NPIECE, Q).transpose(1, 0, 2)
  z_ref[...] = jnp.dot(e, wr_ref[...],
                       preferred_element_type=jnp.float32) + b_ref[...]


def _s3_body(aa_ref, ab_ref, deg_ref, z0_ref, wl_ref, wr_ref, bias_ref,
             y3_ref, z_ref, o_ref):
  inv = 1.0 / jnp.maximum(deg_ref[...], 1.0)
  agg3 = aa_ref[...] + ab_ref[...]
  agg = agg3.transpose(1, 0, 2).reshape(agg3.shape[1], H)
  x = agg * inv + z0_ref[...]
  x1 = jnp.maximum(x, 0.0)
  y = jnp.dot(x1, wl_ref[...], preferred_element_type=jnp.float32)
  y3_ref[...] = y.reshape(y.shape[0], NPIECE, Q).transpose(1, 0, 2)
  z_ref[...] = jnp.dot(x1, wr_ref[...],
                       preferred_element_type=jnp.float32) + bias_ref[...]
  nrm = jnp.sqrt(jnp.sum(x * x, axis=1, keepdims=True))
  o_ref[...] = x / jnp.maximum(nrm, 1e-12)


def _tc_stage1(emb, Wl, Wr, b):
  nb = N // BN
  return pl.pallas_call(
      _s1_body,
      grid=(nb,),
      in_specs=[
          pl.BlockSpec((BN, D), lambda i: (i, 0)),
          pl.BlockSpec((D, H), lambda i: (0, 0)),
          pl.BlockSpec((D, H), lambda i: (0, 0)),
          pl.BlockSpec((1, H), lambda i: (0, 0)),
      ],
      out_specs=[
          pl.BlockSpec((NPIECE, BN, Q), lambda i: (0, i, 0)),
          pl.BlockSpec((BN, H), lambda i: (i, 0)),
      ],
      out_shape=[
          jax.ShapeDtypeStruct((NPIECE, N, Q), jnp.float32),
          jax.ShapeDtypeStruct((N, H), jnp.float32),
      ],
  )(emb, Wl, Wr, b)


def _tc_stage3(aggA, aggB, deg, z0, Wl, Wr, b):
  nb = N // BN
  return pl.pallas_call(
      _s3_body,
      grid=(nb,),
      in_specs=[
          pl.BlockSpec((NPIECE, BN, Q), lambda i: (0, i, 0)),
          pl.BlockSpec((NPIECE, BN, Q), lambda i: (0, i, 0)),
          pl.BlockSpec((BN, 1), lambda i: (i, 0)),
          pl.BlockSpec((BN, H), lambda i: (i, 0)),
          pl.BlockSpec((H, H), lambda i: (0, 0)),
          pl.BlockSpec((H, H), lambda i: (0, 0)),
          pl.BlockSpec((1, H), lambda i: (0, 0)),
      ],
      out_specs=[
          pl.BlockSpec((NPIECE, BN, Q), lambda i: (0, i, 0)),
          pl.BlockSpec((BN, H), lambda i: (i, 0)),
          pl.BlockSpec((BN, H), lambda i: (i, 0)),
      ],
      out_shape=[
          jax.ShapeDtypeStruct((NPIECE, N, Q), jnp.float32),
          jax.ShapeDtypeStruct((N, H), jnp.float32),
          jax.ShapeDtypeStruct((N, H), jnp.float32),
      ],
  )(aggA, aggB, deg, z0, Wl, Wr, b)


def kernel(edge_index, emb, Wl0, Wr0, b0, Wl1, Wr1, b1):
  src = edge_index[0]
  dst = edge_index[1]
  srcp = jnp.pad(src, (0, EPAD - E)).reshape(NROWS, ROW)
  dstp = jnp.pad(dst, (0, EPAD - E),
                 constant_values=jnp.int32(N)).reshape(NROWS, ROW)
  zq_h = jnp.zeros((1568, Q), jnp.float32)
  deg = jax.ops.segment_sum(jnp.ones((E,), jnp.float32), dst,
                            num_segments=N).reshape(N, 1)

  sc_piece = _make_sc_piece()
  y3, z0 = _tc_stage1(emb, Wl0, Wr0, b0.reshape(1, H))

  npiece_rt = NPIECE + jnp.minimum(edge_index[0, 1], 0)
  nlayer_rt = 2 + jnp.minimum(edge_index[0, 0], 0)
  zh = jnp.zeros((H, H), jnp.float32)
  zb = jnp.zeros((1, H), jnp.float32)
  wb1 = b1.reshape(1, H)

  def piece_step(k, carry):
    aggA, aggB, y3c = carry
    yp = lax.dynamic_slice(y3c, (k, 0, 0), (1, N, Q)).reshape(N, Q)
    pa, pb = sc_piece(yp, srcp, dstp, zq_h)
    aggA = lax.dynamic_update_slice(aggA, pa[None], (k, 0, 0))
    aggB = lax.dynamic_update_slice(aggB, pb[None], (k, 0, 0))
    return (aggA, aggB, y3c)

  def layer_cond(state):
    return state[0] < nlayer_rt

  def layer_body(state):
    i, y3c, cz, _ = state
    first = i == 0
    wl = jnp.where(first, Wl1, zh)
    wr = jnp.where(first, Wr1, zh)
    bias = jnp.where(first, wb1, zb)
    agg0 = jnp.zeros((NPIECE, NPAD, Q), jnp.float32)
    aggA, aggB, _ = lax.fori_loop(0, npiece_rt, piece_step,
                                  (agg0, agg0, y3c))
    ny3, nz, outn = _tc_stage3(aggA, aggB, deg, cz, wl, wr, bias)
    return (i + 1, ny3, nz, outn)

  state = (jnp.int32(0), y3, z0, jnp.zeros((N, H), jnp.float32))
  state = lax.while_loop(layer_cond, layer_body, state)
  return state[3]
